# Initial kernel scaffold; baseline (speedup 1.0000x reference)
import jax
import jax.numpy as jnp
from jax.experimental import pallas as pl


def kernel(x, weight):
    raise NotImplementedError("write your pallas kernel here")


# speedup vs baseline: 1.7946x; 1.7946x over previous
import functools

import jax
import jax.numpy as jnp
from jax import lax
from jax.experimental import pallas as pl
from jax.experimental.pallas import tpu as pltpu
from jax.experimental.pallas import tpu_sc as plsc

D = 64
B = 16384 * 50
NC, NS = 2, 16
NW = NC * NS
B_PER_W = B // NW
CHUNK = 512
N_CHUNKS = B_PER_W // CHUNK


def _make_gather():
  mesh = plsc.VectorSubcoreMesh(core_axis_name="c", subcore_axis_name="s")

  @functools.partial(
      pl.kernel,
      mesh=mesh,
      out_type=jax.ShapeDtypeStruct((B, D), jnp.float32),
      compiler_params=pltpu.CompilerParams(use_tc_tiling_on_sc=False),
      scratch_types=[
          pltpu.VMEM((CHUNK,), jnp.int32),
          pltpu.VMEM((CHUNK, D), jnp.float32),
          pltpu.SemaphoreType.DMA,
      ],
  )
  def gather_kernel(idx_hbm, table_hbm, out_hbm, idx_v, rows_v, sem):
    wid = lax.axis_index("s") * NC + lax.axis_index("c")
    base = wid * B_PER_W

    def step(i, carry):
      off = base + i * CHUNK
      pltpu.sync_copy(idx_hbm.at[pl.ds(off, CHUNK)], idx_v)
      pltpu.async_copy(table_hbm.at[idx_v], rows_v, sem).wait()
      pltpu.sync_copy(rows_v, out_hbm.at[pl.ds(off, CHUNK)])
      return carry

    lax.fori_loop(0, N_CHUNKS, step, 0)

  return gather_kernel


_gather = _make_gather()


@jax.jit
def kernel(x, weight):
  idx = x.reshape(-1).astype(jnp.int32)
  out = _gather(idx, weight)
  return out.reshape(x.shape[0], x.shape[1], D)


# speedup vs baseline: 1.8702x; 1.0421x over previous
import functools

import jax
import jax.numpy as jnp
from jax import lax
from jax.experimental import pallas as pl
from jax.experimental.pallas import tpu as pltpu
from jax.experimental.pallas import tpu_sc as plsc

D = 64
B = 16384 * 50
NC, NS = 2, 16
NW = NC * NS
B_PER_W = B // NW
CHUNK = 256
NBUF = 4
N_CHUNKS = B_PER_W // CHUNK
N_GROUPS = N_CHUNKS // NBUF


def _make_gather():
  mesh = plsc.VectorSubcoreMesh(core_axis_name="c", subcore_axis_name="s")

  @functools.partial(
      pl.kernel,
      mesh=mesh,
      out_type=jax.ShapeDtypeStruct((B, D), jnp.float32),
      compiler_params=pltpu.CompilerParams(use_tc_tiling_on_sc=False),
      scratch_types=(
          [pltpu.VMEM((B_PER_W,), jnp.int32)]
          + [pltpu.VMEM((CHUNK, D), jnp.float32) for _ in range(NBUF)]
          + [pltpu.SemaphoreType.DMA for _ in range(2 * NBUF)]
      ),
  )
  def gather_kernel(idx_hbm, table_hbm, out_hbm, idx_v, *rest):
    bufs = rest[:NBUF]
    gsems = rest[NBUF:2 * NBUF]
    ssems = rest[2 * NBUF:]
    wid = lax.axis_index("s") * NC + lax.axis_index("c")
    base = wid * B_PER_W

    pltpu.sync_copy(idx_hbm.at[pl.ds(base, B_PER_W)], idx_v)

    def start_gather(chunk, b):
      off = chunk * CHUNK
      pltpu.async_copy(
          table_hbm.at[idx_v.at[pl.ds(off, CHUNK)]], bufs[b], gsems[b])

    def wait_gather(b):
      pltpu.make_async_copy(
          table_hbm.at[idx_v.at[pl.ds(0, CHUNK)]], bufs[b], gsems[b]).wait()

    def start_store(chunk, b):
      off = base + chunk * CHUNK
      pltpu.async_copy(bufs[b], out_hbm.at[pl.ds(off, CHUNK)], ssems[b])

    def wait_store(b):
      pltpu.make_async_copy(
          bufs[b], out_hbm.at[pl.ds(base, CHUNK)], ssems[b]).wait()

    for b in range(NBUF):
      start_gather(b, b)

    def group_body(gi, carry):
      for b in range(NBUF):
        wait_gather(b)
        start_store(gi * NBUF + b, b)
      for b in range(NBUF):
        wait_store(b)
        start_gather((gi + 1) * NBUF + b, b)
      return carry

    lax.fori_loop(0, N_GROUPS - 1, group_body, 0)

    for b in range(NBUF):
      wait_gather(b)
      start_store((N_GROUPS - 1) * NBUF + b, b)
    for b in range(NBUF):
      wait_store(b)

  return gather_kernel


_gather = _make_gather()


@jax.jit
def kernel(x, weight):
  idx = x.reshape(-1).astype(jnp.int32)
  out = _gather(idx, weight)
  return out.reshape(x.shape[0], x.shape[1], D)


# speedup vs baseline: 2.1608x; 1.1554x over previous
import functools

import jax
import jax.numpy as jnp
from jax import lax
from jax.experimental import pallas as pl
from jax.experimental.pallas import tpu as pltpu
from jax.experimental.pallas import tpu_sc as plsc

V = 1000000
D = 64
B = 16384 * 50
NC, NS = 2, 16
NW = NC * NS
B_PER_W = B // NW
CHUNK = 256
NBUF = 4
N_CHUNKS = B_PER_W // CHUNK
N_GROUPS = N_CHUNKS // NBUF

TCOLS = 512
NT = pl.cdiv(V, 2 * TCOLS)
F = NT * TCOLS
VP = 2 * F


def _linearize_body(lo_ref, hi_ref, out_ref):
  out_ref[:, 0:64] = lo_ref[...].T
  out_ref[:, 64:128] = hi_ref[...].T


_linearize = pl.pallas_call(
    _linearize_body,
    grid=(NT,),
    in_specs=[
        pl.BlockSpec((D, TCOLS), lambda i: (0, i)),
        pl.BlockSpec((D, TCOLS), lambda i: (0, i + NT)),
    ],
    out_specs=pl.BlockSpec((TCOLS, 128), lambda i: (i, 0)),
    out_shape=jax.ShapeDtypeStruct((F, 128), jnp.float32),
)


def _make_gather():
  mesh = plsc.VectorSubcoreMesh(core_axis_name="c", subcore_axis_name="s")

  @functools.partial(
      pl.kernel,
      mesh=mesh,
      out_type=jax.ShapeDtypeStruct((B, D), jnp.float32),
      compiler_params=pltpu.CompilerParams(use_tc_tiling_on_sc=False),
      scratch_types=(
          [pltpu.VMEM((B_PER_W,), jnp.int32)]
          + [pltpu.VMEM((CHUNK, D), jnp.float32) for _ in range(NBUF)]
          + [pltpu.SemaphoreType.DMA for _ in range(2 * NBUF)]
      ),
  )
  def gather_kernel(idx_hbm, table_hbm, out_hbm, idx_v, *rest):
    bufs = rest[:NBUF]
    gsems = rest[NBUF:2 * NBUF]
    ssems = rest[2 * NBUF:]
    wid = lax.axis_index("s") * NC + lax.axis_index("c")
    base = wid * B_PER_W

    pltpu.sync_copy(idx_hbm.at[pl.ds(base, B_PER_W)], idx_v)

    def start_gather(chunk, b):
      off = chunk * CHUNK
      pltpu.async_copy(
          table_hbm.at[idx_v.at[pl.ds(off, CHUNK)]], bufs[b], gsems[b])

    def wait_gather(b):
      pltpu.make_async_copy(
          table_hbm.at[idx_v.at[pl.ds(0, CHUNK)]], bufs[b], gsems[b]).wait()

    def start_store(chunk, b):
      off = base + chunk * CHUNK
      pltpu.async_copy(bufs[b], out_hbm.at[pl.ds(off, CHUNK)], ssems[b])

    def wait_store(b):
      pltpu.make_async_copy(
          bufs[b], out_hbm.at[pl.ds(base, CHUNK)], ssems[b]).wait()

    for b in range(NBUF):
      start_gather(b, b)

    def group_body(gi, carry):
      for b in range(NBUF):
        wait_gather(b)
        start_store(gi * NBUF + b, b)
      for b in range(NBUF):
        wait_store(b)
        start_gather((gi + 1) * NBUF + b, b)
      return carry

    lax.fori_loop(0, N_GROUPS - 1, group_body, 0)

    for b in range(NBUF):
      wait_gather(b)
      start_store((N_GROUPS - 1) * NBUF + b, b)
    for b in range(NBUF):
      wait_store(b)

  return gather_kernel


_gather = _make_gather()

BBLK = 128


def _to_batch_minor_body(r_ref, out_ref):
  r3 = r_ref[...].reshape(BBLK, 25, 128)
  for t in range(25):
    mt = r3[:, t, :].T
    out_ref[2 * t, :, :] = mt[0:64, :]
    out_ref[2 * t + 1, :, :] = mt[64:128, :]


_to_batch_minor = pl.pallas_call(
    _to_batch_minor_body,
    grid=(16384 // BBLK,),
    in_specs=[pl.BlockSpec((BBLK * 25, 128), lambda i: (i, 0))],
    out_specs=pl.BlockSpec((50, 64, BBLK), lambda i: (0, 0, i)),
    out_shape=jax.ShapeDtypeStruct((50, 64, 16384), jnp.float32),
)


@jax.jit
def kernel(x, weight):
  idx = x.reshape(-1).astype(jnp.int32)
  idx = jnp.where(idx < F, 2 * idx, 2 * (idx - F) + 1)
  wt = weight.T
  table2 = _linearize(wt, wt)
  table = table2.reshape(VP, D)
  rows = _gather(idx, table)
  rows2 = rows.reshape(B // 2, 128)
  out_t = _to_batch_minor(rows2)
  return jnp.transpose(out_t, (2, 0, 1))


# speedup vs baseline: 3.7715x; 1.7454x over previous
import functools

import jax
import jax.numpy as jnp
from jax import lax
from jax.experimental import pallas as pl
from jax.experimental.pallas import tpu as pltpu
from jax.experimental.pallas import tpu_sc as plsc

V = 1000000
D = 64
B = 16384 * 50
NC, NS = 2, 16
NW = NC * NS
B_PER_W = B // NW
CHUNK = 256
NBUF = 4
N_CHUNKS = B_PER_W // CHUNK
N_GROUPS = N_CHUNKS // NBUF

TCOLS = 8960
NT = pl.cdiv(V, 2 * TCOLS)
F = NT * TCOLS
VP = 2 * F


def _linearize_body(lo_ref, hi_ref, out_ref):
  out_ref[:, 0:64] = lo_ref[...].T
  out_ref[:, 64:128] = hi_ref[...].T


_linearize = pl.pallas_call(
    _linearize_body,
    grid=(NT,),
    in_specs=[
        pl.BlockSpec((D, TCOLS), lambda i: (0, i)),
        pl.BlockSpec((D, TCOLS), lambda i: (0, i + NT)),
    ],
    out_specs=pl.BlockSpec((TCOLS, 128), lambda i: (i, 0)),
    out_shape=jax.ShapeDtypeStruct((F, 128), jnp.float32),
)


def _make_gather():
  mesh = plsc.VectorSubcoreMesh(core_axis_name="c", subcore_axis_name="s")

  @functools.partial(
      pl.kernel,
      mesh=mesh,
      out_type=jax.ShapeDtypeStruct((B, D), jnp.float32),
      compiler_params=pltpu.CompilerParams(use_tc_tiling_on_sc=False),
      scratch_types=(
          [pltpu.VMEM((B_PER_W,), jnp.int32)]
          + [pltpu.VMEM((CHUNK, D), jnp.float32) for _ in range(NBUF)]
          + [pltpu.SemaphoreType.DMA for _ in range(2 * NBUF)]
      ),
  )
  def gather_kernel(idx_hbm, table_hbm, out_hbm, idx_v, *rest):
    bufs = rest[:NBUF]
    gsems = rest[NBUF:2 * NBUF]
    ssems = rest[2 * NBUF:]
    wid = lax.axis_index("s") * NC + lax.axis_index("c")
    base = wid * B_PER_W

    pltpu.sync_copy(idx_hbm.at[pl.ds(base, B_PER_W)], idx_v)

    def start_gather(chunk, b):
      off = chunk * CHUNK
      pltpu.async_copy(
          table_hbm.at[idx_v.at[pl.ds(off, CHUNK)]], bufs[b], gsems[b])

    def wait_gather(b):
      pltpu.make_async_copy(
          table_hbm.at[idx_v.at[pl.ds(0, CHUNK)]], bufs[b], gsems[b]).wait()

    def start_store(chunk, b):
      off = base + chunk * CHUNK
      pltpu.async_copy(bufs[b], out_hbm.at[pl.ds(off, CHUNK)], ssems[b])

    def wait_store(b):
      pltpu.make_async_copy(
          bufs[b], out_hbm.at[pl.ds(base, CHUNK)], ssems[b]).wait()

    for b in range(NBUF):
      start_gather(b, b)

    def group_body(gi, carry):
      for b in range(NBUF):
        wait_gather(b)
        start_store(gi * NBUF + b, b)
      for b in range(NBUF):
        wait_store(b)
        start_gather((gi + 1) * NBUF + b, b)
      return carry

    lax.fori_loop(0, N_GROUPS - 1, group_body, 0)

    for b in range(NBUF):
      wait_gather(b)
      start_store((N_GROUPS - 1) * NBUF + b, b)
    for b in range(NBUF):
      wait_store(b)

  return gather_kernel


_gather = _make_gather()

BBLK = 128


def _to_batch_minor_body(r_ref, out_ref):
  r3 = r_ref[...].reshape(BBLK, 25, 128)
  for t in range(25):
    mt = r3[:, t, :].T
    out_ref[2 * t, :, :] = mt[0:64, :]
    out_ref[2 * t + 1, :, :] = mt[64:128, :]


_to_batch_minor = pl.pallas_call(
    _to_batch_minor_body,
    grid=(16384 // BBLK,),
    in_specs=[pl.BlockSpec((BBLK * 25, 128), lambda i: (i, 0))],
    out_specs=pl.BlockSpec((50, 64, BBLK), lambda i: (0, 0, i)),
    out_shape=jax.ShapeDtypeStruct((50, 64, 16384), jnp.float32),
)


@jax.jit
def kernel(x, weight):
  idx = x.reshape(-1).astype(jnp.int32)
  idx = jnp.where(idx < F, 2 * idx, 2 * (idx - F) + 1)
  wt = weight.T
  table2 = _linearize(wt, wt)
  table = table2.reshape(VP, D)
  rows = _gather(idx, table)
  rows2 = rows.reshape(B // 2, 128)
  out_t = _to_batch_minor(rows2)
  return jnp.transpose(out_t, (2, 0, 1))


# speedup vs baseline: 4.0787x; 1.0814x over previous
import functools

import jax
import jax.numpy as jnp
from jax import lax
from jax.experimental import pallas as pl
from jax.experimental.pallas import tpu as pltpu
from jax.experimental.pallas import tpu_sc as plsc

V = 1000000
D = 64
B = 16384 * 50
NC, NS = 2, 16
NW = NC * NS
NCH = 4
BC = B // NCH
B_PER_W = BC // NW
CHUNK = 256
NBUF = 5
N_CHUNKS = B_PER_W // CHUNK
N_GROUPS = N_CHUNKS // NBUF

TCOLS = 8960
NT = pl.cdiv(V, 2 * TCOLS)
F = NT * TCOLS
VP = 2 * F


def _linearize_body(lo_ref, hi_ref, out_ref):
  out_ref[:, 0:64] = lo_ref[...].T
  out_ref[:, 64:128] = hi_ref[...].T


_linearize = pl.pallas_call(
    _linearize_body,
    grid=(NT,),
    in_specs=[
        pl.BlockSpec((D, TCOLS), lambda i: (0, i)),
        pl.BlockSpec((D, TCOLS), lambda i: (0, i + NT)),
    ],
    out_specs=pl.BlockSpec((TCOLS, 128), lambda i: (i, 0)),
    out_shape=jax.ShapeDtypeStruct((F, 128), jnp.float32),
)


def _make_gather():
  mesh = plsc.VectorSubcoreMesh(core_axis_name="c", subcore_axis_name="s")

  @functools.partial(
      pl.kernel,
      mesh=mesh,
      out_type=jax.ShapeDtypeStruct((BC, D), jnp.float32),
      compiler_params=pltpu.CompilerParams(use_tc_tiling_on_sc=False),
      scratch_types=(
          [pltpu.VMEM((B_PER_W,), jnp.int32)]
          + [pltpu.VMEM((CHUNK, D), jnp.float32) for _ in range(NBUF)]
          + [pltpu.SemaphoreType.DMA for _ in range(2 * NBUF)]
      ),
  )
  def gather_kernel(idx_hbm, table_hbm, out_hbm, idx_v, *rest):
    bufs = rest[:NBUF]
    gsems = rest[NBUF:2 * NBUF]
    ssems = rest[2 * NBUF:]
    wid = lax.axis_index("s") * NC + lax.axis_index("c")
    base = wid * B_PER_W

    pltpu.sync_copy(idx_hbm.at[pl.ds(base, B_PER_W)], idx_v)

    def start_gather(chunk, b):
      off = chunk * CHUNK
      pltpu.async_copy(
          table_hbm.at[idx_v.at[pl.ds(off, CHUNK)]], bufs[b], gsems[b])

    def wait_gather(b):
      pltpu.make_async_copy(
          table_hbm.at[idx_v.at[pl.ds(0, CHUNK)]], bufs[b], gsems[b]).wait()

    def start_store(chunk, b):
      off = base + chunk * CHUNK
      pltpu.async_copy(bufs[b], out_hbm.at[pl.ds(off, CHUNK)], ssems[b])

    def wait_store(b):
      pltpu.make_async_copy(
          bufs[b], out_hbm.at[pl.ds(base, CHUNK)], ssems[b]).wait()

    for b in range(NBUF):
      start_gather(b, b)

    def group_body(gi, carry):
      for b in range(NBUF):
        wait_gather(b)
        start_store(gi * NBUF + b, b)
      for b in range(NBUF):
        wait_store(b)
        start_gather((gi + 1) * NBUF + b, b)
      return carry

    lax.fori_loop(0, N_GROUPS - 1, group_body, 0)

    for b in range(NBUF):
      wait_gather(b)
      start_store((N_GROUPS - 1) * NBUF + b, b)
    for b in range(NBUF):
      wait_store(b)

  return gather_kernel


_gather = _make_gather()

BBLK = 128


def _to_batch_minor_body(r_ref, out_ref):
  r3 = r_ref[...].reshape(BBLK, 25, 128)
  for t in range(25):
    mt = r3[:, t, :].T
    out_ref[2 * t, :, :] = mt[0:64, :]
    out_ref[2 * t + 1, :, :] = mt[64:128, :]


NB = 16384 // NCH // BBLK


def _make_to_batch_minor(c, aliased):
  if aliased:
    def body(r_ref, acc_ref, out_ref):
      _to_batch_minor_body(r_ref, out_ref)
    in_specs = [
        pl.BlockSpec((BBLK * 25, 128), lambda i: (i, 0)),
        pl.BlockSpec(memory_space=pl.ANY),
    ]
    alias = {1: 0}
  else:
    body = _to_batch_minor_body
    in_specs = [pl.BlockSpec((BBLK * 25, 128), lambda i: (i, 0))]
    alias = {}
  return pl.pallas_call(
      body,
      grid=(NB,),
      in_specs=in_specs,
      out_specs=pl.BlockSpec((50, 64, BBLK), lambda i: (0, 0, i + c * NB)),
      out_shape=jax.ShapeDtypeStruct((50, 64, 16384), jnp.float32),
      input_output_aliases=alias,
  )


_to_batch_minor_chunks = [_make_to_batch_minor(c, c > 0) for c in range(NCH)]


@jax.jit
def kernel(x, weight):
  idx = x.reshape(-1).astype(jnp.int32)
  idx = jnp.where(idx < F, 2 * idx, 2 * (idx - F) + 1)
  wt = weight.T
  table2 = _linearize(wt, wt)
  table = table2.reshape(VP, D)
  out_t = None
  for c in range(NCH):
    rows = _gather(idx[c * BC:(c + 1) * BC], table)
    rows2 = rows.reshape(BC // 2, 128)
    if c == 0:
      out_t = _to_batch_minor_chunks[0](rows2)
    else:
      out_t = _to_batch_minor_chunks[c](rows2, out_t)
  return jnp.transpose(out_t, (2, 0, 1))


# speedup vs baseline: 4.3127x; 1.0574x over previous
import functools

import jax
import jax.numpy as jnp
from jax import lax
from jax.experimental import pallas as pl
from jax.experimental.pallas import tpu as pltpu
from jax.experimental.pallas import tpu_sc as plsc

V = 1000000
D = 64
B = 16384 * 50
NC, NS = 2, 16
NW = NC * NS
NCH = 4
BC = B // NCH
B_PER_W = BC // NW
CHUNK = 256
NBUF = 5
N_CHUNKS = B_PER_W // CHUNK
N_GROUPS = N_CHUNKS // NBUF

TCOLS = 12288
NT = pl.cdiv(V, 2 * TCOLS)
F = NT * TCOLS
VP = 2 * F


def _linearize_body(lo_ref, hi_ref, out_ref):
  out_ref[:, 0:64] = lo_ref[...].T
  out_ref[:, 64:128] = hi_ref[...].T


_linearize = pl.pallas_call(
    _linearize_body,
    grid=(NT,),
    in_specs=[
        pl.BlockSpec((D, TCOLS), lambda i: (0, i)),
        pl.BlockSpec((D, TCOLS), lambda i: (0, i + NT)),
    ],
    out_specs=pl.BlockSpec((TCOLS, 128), lambda i: (i, 0)),
    out_shape=jax.ShapeDtypeStruct((F, 128), jnp.float32),
)


def _make_gather():
  mesh = plsc.VectorSubcoreMesh(core_axis_name="c", subcore_axis_name="s")

  @functools.partial(
      pl.kernel,
      mesh=mesh,
      out_type=jax.ShapeDtypeStruct((BC, D), jnp.float32),
      compiler_params=pltpu.CompilerParams(use_tc_tiling_on_sc=False),
      scratch_types=(
          [pltpu.VMEM((B_PER_W,), jnp.int32)]
          + [pltpu.VMEM((CHUNK, D), jnp.float32) for _ in range(NBUF)]
          + [pltpu.SemaphoreType.DMA for _ in range(2 * NBUF)]
      ),
  )
  def gather_kernel(idx_hbm, table_hbm, out_hbm, idx_v, *rest):
    bufs = rest[:NBUF]
    gsems = rest[NBUF:2 * NBUF]
    ssems = rest[2 * NBUF:]
    wid = lax.axis_index("s") * NC + lax.axis_index("c")
    base = wid * B_PER_W

    pltpu.sync_copy(idx_hbm.at[pl.ds(base, B_PER_W)], idx_v)

    def start_gather(chunk, b):
      off = chunk * CHUNK
      pltpu.async_copy(
          table_hbm.at[idx_v.at[pl.ds(off, CHUNK)]], bufs[b], gsems[b])

    def wait_gather(b):
      pltpu.make_async_copy(
          table_hbm.at[idx_v.at[pl.ds(0, CHUNK)]], bufs[b], gsems[b]).wait()

    def start_store(chunk, b):
      off = base + chunk * CHUNK
      pltpu.async_copy(bufs[b], out_hbm.at[pl.ds(off, CHUNK)], ssems[b])

    def wait_store(b):
      pltpu.make_async_copy(
          bufs[b], out_hbm.at[pl.ds(base, CHUNK)], ssems[b]).wait()

    for b in range(NBUF):
      start_gather(b, b)

    def group_body(gi, carry):
      for b in range(NBUF):
        wait_gather(b)
        start_store(gi * NBUF + b, b)
      for b in range(NBUF):
        wait_store(b)
        start_gather((gi + 1) * NBUF + b, b)
      return carry

    lax.fori_loop(0, N_GROUPS - 1, group_body, 0)

    for b in range(NBUF):
      wait_gather(b)
      start_store((N_GROUPS - 1) * NBUF + b, b)
    for b in range(NBUF):
      wait_store(b)

  return gather_kernel


_gather = _make_gather()

BBLK = 256


def _to_batch_minor_body(r_ref, out_ref):
  r3 = r_ref[...].reshape(BBLK, 25, 128)
  for t in range(25):
    mt = r3[:, t, :].T
    out_ref[2 * t, :, :] = mt[0:64, :]
    out_ref[2 * t + 1, :, :] = mt[64:128, :]


NB = 16384 // NCH // BBLK


def _make_to_batch_minor(c, aliased):
  if aliased:
    def body(r_ref, acc_ref, out_ref):
      _to_batch_minor_body(r_ref, out_ref)
    in_specs = [
        pl.BlockSpec((BBLK * 25, 128), lambda i: (i, 0)),
        pl.BlockSpec(memory_space=pl.ANY),
    ]
    alias = {1: 0}
  else:
    body = _to_batch_minor_body
    in_specs = [pl.BlockSpec((BBLK * 25, 128), lambda i: (i, 0))]
    alias = {}
  return pl.pallas_call(
      body,
      grid=(NB,),
      in_specs=in_specs,
      out_specs=pl.BlockSpec((50, 64, BBLK), lambda i: (0, 0, i + c * NB)),
      out_shape=jax.ShapeDtypeStruct((50, 64, 16384), jnp.float32),
      input_output_aliases=alias,
  )


_to_batch_minor_chunks = [_make_to_batch_minor(c, c > 0) for c in range(NCH)]


@jax.jit
def kernel(x, weight):
  idx = x.reshape(-1).astype(jnp.int32)
  idx = jnp.where(idx < F, 2 * idx, 2 * (idx - F) + 1)
  wt = weight.T
  table2 = _linearize(wt, wt)
  table = table2.reshape(VP, D)
  out_t = None
  for c in range(NCH):
    rows = _gather(idx[c * BC:(c + 1) * BC], table)
    rows2 = rows.reshape(BC // 2, 128)
    if c == 0:
      out_t = _to_batch_minor_chunks[0](rows2)
    else:
      out_t = _to_batch_minor_chunks[c](rows2, out_t)
  return jnp.transpose(out_t, (2, 0, 1))


# speedup vs baseline: 4.3219x; 1.0021x over previous
import functools

import jax
import jax.numpy as jnp
from jax import lax
from jax.experimental import pallas as pl
from jax.experimental.pallas import tpu as pltpu
from jax.experimental.pallas import tpu_sc as plsc

V = 1000000
D = 64
B = 16384 * 50
NC, NS = 2, 16
NW = NC * NS
NCH = 4
BC = B // NCH
B_PER_W = BC // NW
CHUNK = 256
NBUF = 5
N_CHUNKS = B_PER_W // CHUNK
N_GROUPS = N_CHUNKS // NBUF

TCOLS = 16384
NT = pl.cdiv(V, 2 * TCOLS)
F = NT * TCOLS
VP = 2 * F


def _linearize_body(lo_ref, hi_ref, out_ref):
  out_ref[:, 0:64] = lo_ref[...].T
  out_ref[:, 64:128] = hi_ref[...].T


_linearize = pl.pallas_call(
    _linearize_body,
    grid=(NT,),
    in_specs=[
        pl.BlockSpec((D, TCOLS), lambda i: (0, i)),
        pl.BlockSpec((D, TCOLS), lambda i: (0, i + NT)),
    ],
    out_specs=pl.BlockSpec((TCOLS, 128), lambda i: (i, 0)),
    out_shape=jax.ShapeDtypeStruct((F, 128), jnp.float32),
)


def _make_gather():
  mesh = plsc.VectorSubcoreMesh(core_axis_name="c", subcore_axis_name="s")

  @functools.partial(
      pl.kernel,
      mesh=mesh,
      out_type=jax.ShapeDtypeStruct((BC, D), jnp.float32),
      compiler_params=pltpu.CompilerParams(use_tc_tiling_on_sc=False),
      scratch_types=(
          [pltpu.VMEM((B_PER_W,), jnp.int32)]
          + [pltpu.VMEM((CHUNK, D), jnp.float32) for _ in range(NBUF)]
          + [pltpu.SemaphoreType.DMA for _ in range(2 * NBUF)]
      ),
  )
  def gather_kernel(idx_hbm, table_hbm, out_hbm, idx_v, *rest):
    bufs = rest[:NBUF]
    gsems = rest[NBUF:2 * NBUF]
    ssems = rest[2 * NBUF:]
    wid = lax.axis_index("s") * NC + lax.axis_index("c")
    base = wid * B_PER_W

    pltpu.sync_copy(idx_hbm.at[pl.ds(base, B_PER_W)], idx_v)

    def start_gather(chunk, b):
      off = chunk * CHUNK
      pltpu.async_copy(
          table_hbm.at[idx_v.at[pl.ds(off, CHUNK)]], bufs[b], gsems[b])

    def wait_gather(b):
      pltpu.make_async_copy(
          table_hbm.at[idx_v.at[pl.ds(0, CHUNK)]], bufs[b], gsems[b]).wait()

    def start_store(chunk, b):
      off = base + chunk * CHUNK
      pltpu.async_copy(bufs[b], out_hbm.at[pl.ds(off, CHUNK)], ssems[b])

    def wait_store(b):
      pltpu.make_async_copy(
          bufs[b], out_hbm.at[pl.ds(base, CHUNK)], ssems[b]).wait()

    for b in range(NBUF):
      start_gather(b, b)

    def group_body(gi, carry):
      for b in range(NBUF):
        wait_gather(b)
        start_store(gi * NBUF + b, b)
      for b in range(NBUF):
        wait_store(b)
        start_gather((gi + 1) * NBUF + b, b)
      return carry

    lax.fori_loop(0, N_GROUPS - 1, group_body, 0)

    for b in range(NBUF):
      wait_gather(b)
      start_store((N_GROUPS - 1) * NBUF + b, b)
    for b in range(NBUF):
      wait_store(b)

  return gather_kernel


_gather = _make_gather()

BBLK = 256


def _to_batch_minor_body(r_ref, out_ref):
  r3 = r_ref[...].reshape(BBLK, 25, 128)
  for t in range(25):
    mt = r3[:, t, :].T
    out_ref[2 * t, :, :] = mt[0:64, :]
    out_ref[2 * t + 1, :, :] = mt[64:128, :]


NB = 16384 // NCH // BBLK


def _make_to_batch_minor(c, aliased):
  if aliased:
    def body(r_ref, acc_ref, out_ref):
      _to_batch_minor_body(r_ref, out_ref)
    in_specs = [
        pl.BlockSpec((BBLK * 25, 128), lambda i: (i, 0)),
        pl.BlockSpec(memory_space=pl.ANY),
    ]
    alias = {1: 0}
  else:
    body = _to_batch_minor_body
    in_specs = [pl.BlockSpec((BBLK * 25, 128), lambda i: (i, 0))]
    alias = {}
  return pl.pallas_call(
      body,
      grid=(NB,),
      in_specs=in_specs,
      out_specs=pl.BlockSpec((50, 64, BBLK), lambda i: (0, 0, i + c * NB)),
      out_shape=jax.ShapeDtypeStruct((50, 64, 16384), jnp.float32),
      input_output_aliases=alias,
  )


_to_batch_minor_chunks = [_make_to_batch_minor(c, c > 0) for c in range(NCH)]


@jax.jit
def kernel(x, weight):
  idx = x.reshape(-1).astype(jnp.int32)
  idx = jnp.where(idx < F, 2 * idx, 2 * (idx - F) + 1)
  wt = weight.T
  table2 = _linearize(wt, wt)
  table = table2.reshape(VP, D)
  out_t = None
  for c in range(NCH):
    rows = _gather(idx[c * BC:(c + 1) * BC], table)
    rows2 = rows.reshape(BC // 2, 128)
    if c == 0:
      out_t = _to_batch_minor_chunks[0](rows2)
    else:
      out_t = _to_batch_minor_chunks[c](rows2, out_t)
  return jnp.transpose(out_t, (2, 0, 1))


# speedup vs baseline: 4.3951x; 1.0169x over previous
import functools

import jax
import jax.numpy as jnp
from jax import lax
from jax.experimental import pallas as pl
from jax.experimental.pallas import tpu as pltpu
from jax.experimental.pallas import tpu_sc as plsc

V = 1000000
D = 64
B = 16384 * 50
NC, NS = 2, 16
NW = NC * NS
NCH = 4
BC = B // NCH
B_PER_W = BC // NW
CHUNK = 256
NBUF = 5
N_CHUNKS = B_PER_W // CHUNK
N_GROUPS = N_CHUNKS // NBUF

TCOLS = 16384
NT = pl.cdiv(V, 2 * TCOLS)
F = NT * TCOLS
VP = 2 * F


def _linearize_body(lo_ref, hi_ref, out_ref):
  out_ref[:, 0:64] = lo_ref[...].T
  out_ref[:, 64:128] = hi_ref[...].T


_linearize = pl.pallas_call(
    _linearize_body,
    grid=(NT,),
    in_specs=[
        pl.BlockSpec((D, TCOLS), lambda i: (0, i)),
        pl.BlockSpec((D, TCOLS), lambda i: (0, i + NT)),
    ],
    out_specs=pl.BlockSpec((TCOLS, 128), lambda i: (i, 0)),
    out_shape=jax.ShapeDtypeStruct((F, 128), jnp.float32),
)


def _make_gather():
  mesh = plsc.VectorSubcoreMesh(core_axis_name="c", subcore_axis_name="s")

  @functools.partial(
      pl.kernel,
      mesh=mesh,
      out_type=jax.ShapeDtypeStruct((BC, D), jnp.float32),
      compiler_params=pltpu.CompilerParams(use_tc_tiling_on_sc=False),
      scratch_types=(
          [pltpu.VMEM((B_PER_W,), jnp.int32)]
          + [pltpu.VMEM((CHUNK, D), jnp.float32) for _ in range(NBUF)]
          + [pltpu.SemaphoreType.DMA for _ in range(2 * NBUF)]
      ),
  )
  def gather_kernel(idx_hbm, table_hbm, out_hbm, idx_v, *rest):
    bufs = rest[:NBUF]
    gsems = rest[NBUF:2 * NBUF]
    ssems = rest[2 * NBUF:]
    wid = lax.axis_index("s") * NC + lax.axis_index("c")
    base = wid * B_PER_W

    pltpu.sync_copy(idx_hbm.at[pl.ds(base, B_PER_W)], idx_v)

    def start_gather(chunk, b):
      off = chunk * CHUNK
      pltpu.async_copy(
          table_hbm.at[idx_v.at[pl.ds(off, CHUNK)]], bufs[b], gsems[b])

    def wait_gather(b):
      pltpu.make_async_copy(
          table_hbm.at[idx_v.at[pl.ds(0, CHUNK)]], bufs[b], gsems[b]).wait()

    def start_store(chunk, b):
      off = base + chunk * CHUNK
      pltpu.async_copy(bufs[b], out_hbm.at[pl.ds(off, CHUNK)], ssems[b])

    def wait_store(b):
      pltpu.make_async_copy(
          bufs[b], out_hbm.at[pl.ds(base, CHUNK)], ssems[b]).wait()

    for b in range(NBUF):
      start_gather(b, b)

    def group_body(gi, carry):
      for b in range(NBUF):
        wait_gather(b)
        start_store(gi * NBUF + b, b)
      for b in range(NBUF):
        wait_store(b)
        start_gather((gi + 1) * NBUF + b, b)
      return carry

    lax.fori_loop(0, N_GROUPS - 1, group_body, 0)

    for b in range(NBUF):
      wait_gather(b)
      start_store((N_GROUPS - 1) * NBUF + b, b)
    for b in range(NBUF):
      wait_store(b)

  return gather_kernel


_gather = _make_gather()

BBLK = 512


def _to_batch_minor_body(r_ref, out_ref):
  r3 = r_ref[...].reshape(BBLK, 25, 128)
  for t in range(25):
    mt = r3[:, t, :].T
    out_ref[2 * t, :, :] = mt[0:64, :]
    out_ref[2 * t + 1, :, :] = mt[64:128, :]


NB = 16384 // NCH // BBLK


def _make_to_batch_minor(c, aliased):
  if aliased:
    def body(r_ref, acc_ref, out_ref):
      _to_batch_minor_body(r_ref, out_ref)
    in_specs = [
        pl.BlockSpec((BBLK * 25, 128), lambda i: (i, 0)),
        pl.BlockSpec(memory_space=pl.ANY),
    ]
    alias = {1: 0}
  else:
    body = _to_batch_minor_body
    in_specs = [pl.BlockSpec((BBLK * 25, 128), lambda i: (i, 0))]
    alias = {}
  return pl.pallas_call(
      body,
      grid=(NB,),
      in_specs=in_specs,
      out_specs=pl.BlockSpec((50, 64, BBLK), lambda i: (0, 0, i + c * NB)),
      out_shape=jax.ShapeDtypeStruct((50, 64, 16384), jnp.float32),
      input_output_aliases=alias,
  )


_to_batch_minor_chunks = [_make_to_batch_minor(c, c > 0) for c in range(NCH)]


@jax.jit
def kernel(x, weight):
  idx = x.reshape(-1).astype(jnp.int32)
  idx = jnp.where(idx < F, 2 * idx, 2 * (idx - F) + 1)
  wt = weight.T
  table2 = _linearize(wt, wt)
  table = table2.reshape(VP, D)
  out_t = None
  for c in range(NCH):
    rows = _gather(idx[c * BC:(c + 1) * BC], table)
    rows2 = rows.reshape(BC // 2, 128)
    if c == 0:
      out_t = _to_batch_minor_chunks[0](rows2)
    else:
      out_t = _to_batch_minor_chunks[c](rows2, out_t)
  return jnp.transpose(out_t, (2, 0, 1))
